# baseline (speedup 1.0000x reference)
import functools

import jax
import jax.numpy as jnp
from jax import lax
from jax.experimental import pallas as pl
from jax.experimental.pallas import tpu as pltpu
from jax.experimental.pallas import tpu_sc as plsc

_SC_CORES = 2
_SC_SUBCORES = 16


def _gelu(x):
    return 0.5 * x * (1.0 + jax.lax.erf(x * (2.0 ** -0.5)))


def _dot(a, b):
    return jax.lax.dot_general(a, b, (((1,), (0,)), ((), ())),
                               preferred_element_type=jnp.float32)


def _node_mlp_kernel(hid, fin, x8, u8, w1, b1, w2, b2, w3, b3, out, psum):
    h = _gelu(_dot(fin[...], w1[...]) + b1[...])
    h = _gelu(_dot(h, w2[...]) + b2[...])
    nf = _dot(h, w3[...]) + b3[...]
    nblk = nf.shape[0]
    x8b = x8[...]
    ndu = _dot(x8b, u8[...])
    xn2 = jnp.sum(x8b * x8b, axis=1, keepdims=True)
    pad = jnp.zeros((nblk, 123), jnp.float32)
    out[...] = jnp.concatenate([nf, ndu, xn2, x8b[:, :3], pad], axis=1)
    psum[...] = jnp.sum(nf, axis=0, keepdims=True).reshape(1, 1, hid)


def _node_mlp(fin, x8, u8, node_params, nblk):
    n, cin = fin.shape
    (w1, b1), (w2, b2), (w3, b3) = node_params
    hid = w3.shape[1]
    hp = hid + 128
    grid = n // nblk
    full = lambda *s: pl.BlockSpec(s, lambda i: tuple(0 for _ in s))
    out, psum = pl.pallas_call(
        functools.partial(_node_mlp_kernel, hid),
        grid=(grid,),
        in_specs=[
            pl.BlockSpec((nblk, cin), lambda i: (i, 0)),
            pl.BlockSpec((nblk, 8), lambda i: (i, 0)),
            full(8, 1),
            full(cin, hid), full(1, hid),
            full(hid, hid), full(1, hid),
            full(hid, hid), full(1, hid),
        ],
        out_specs=[
            pl.BlockSpec((nblk, hp), lambda i: (i, 0)),
            pl.BlockSpec((1, 1, hid), lambda i: (i, 0, 0)),
        ],
        out_shape=[
            jax.ShapeDtypeStruct((n, hp), jnp.float32),
            jax.ShapeDtypeStruct((grid, 1, hid), jnp.float32),
        ],
    )(fin, x8, u8, w1, b1.reshape(1, -1), w2, b2.reshape(1, -1), w3,
      b3.reshape(1, -1))
    return out, psum.reshape(grid, hid)


def _dist_kernel(kk, a8, xT8, u8, uincl, uexcl, enc, adu):
    ab = a8[...]
    xt = xT8[...]
    an2 = jnp.sum(ab * ab, axis=1, keepdims=True)
    xn2 = jnp.sum(xt * xt, axis=0, keepdims=True)
    prod = _dot(ab, xt)
    d2b = an2 + xn2 - 2.0 * prod
    adu[...] = _dot(ab, u8[...])

    bits = lax.bitcast_convert_type(d2b, jnp.uint32)
    sign = bits >> jnp.uint32(31)
    key = bits ^ (sign * jnp.uint32(0xFFFFFFFF) | jnp.uint32(0x80000000))
    p = jnp.zeros(d2b.shape[:1] + (1,), jnp.uint32)
    for b in range(31, -1, -1):
        cand = p | jnp.uint32((1 << b) - 1)
        cnt = jnp.sum((key <= cand).astype(jnp.int32), axis=1, keepdims=True)
        p = jnp.where(cnt >= kk, p, p | jnp.uint32(1 << b))
    vbits = jnp.where(p >= jnp.uint32(0x80000000),
                      p ^ jnp.uint32(0x80000000), ~p)
    thr_f = lax.bitcast_convert_type(vbits, jnp.float32)
    lt = d2b < thr_f
    eq = d2b == thr_f
    cnt_lt = jnp.sum(jnp.where(lt, 1.0, 0.0), axis=1, keepdims=True)
    budf = kk - cnt_lt

    n = d2b.shape[1]
    nch = n // 128
    ui = uincl[...]
    ue = uexcl[...]
    one = jnp.float32(1.0)
    zero = jnp.float32(0.0)
    peq = []
    for c in range(nch):
        sl = slice(c * 128, (c + 1) * 128)
        peq.append(_dot(jnp.where(eq[:, sl], one, zero), ui))
    tot_eq = jnp.concatenate([q[:, 127:128] for q in peq], axis=1)
    off_eq = _dot(tot_eq, ue)
    takes = []
    ptk = []
    for c in range(nch):
        sl = slice(c * 128, (c + 1) * 128)
        tie_rank = peq[c] + off_eq[:, c:c + 1]
        take_c = lt[:, sl] | (eq[:, sl] & (tie_rank <= budf))
        takes.append(take_c)
        ptk.append(_dot(jnp.where(take_c, one, zero), ui))
    tot_tk = jnp.concatenate([q[:, 127:128] for q in ptk], axis=1)
    off_tk = _dot(tot_tk, ue)
    for c in range(nch):
        pos = ptk[c] - 1.0 + off_tk[:, c:c + 1]
        enc[:, c * 128:(c + 1) * 128] = jnp.where(
            takes[c], pos.astype(jnp.int32), jnp.int32(-1))


def _distances(a8, xT8, u8, ablk, kk):
    na = a8.shape[0]
    n = xT8.shape[1]
    grid = na // ablk
    nch = n // 128
    ii = lax.broadcasted_iota(jnp.int32, (128, 128), 0)
    jj = lax.broadcasted_iota(jnp.int32, (128, 128), 1)
    uincl = jnp.where(ii <= jj, 1.0, 0.0).astype(jnp.float32)
    i2 = lax.broadcasted_iota(jnp.int32, (nch, nch), 0)
    j2 = lax.broadcasted_iota(jnp.int32, (nch, nch), 1)
    uexcl = jnp.where(i2 < j2, 1.0, 0.0).astype(jnp.float32)
    enc, adu = pl.pallas_call(
        functools.partial(_dist_kernel, kk),
        grid=(grid,),
        in_specs=[
            pl.BlockSpec((ablk, 8), lambda i: (i, 0)),
            pl.BlockSpec((8, n), lambda i: (0, 0)),
            pl.BlockSpec((8, 1), lambda i: (0, 0)),
            pl.BlockSpec((128, 128), lambda i: (0, 0)),
            pl.BlockSpec((nch, nch), lambda i: (0, 0)),
        ],
        out_specs=[
            pl.BlockSpec((ablk, n), lambda i: (i, 0)),
            pl.BlockSpec((ablk, 1), lambda i: (i, 0)),
        ],
        out_shape=[
            jax.ShapeDtypeStruct((na, n), jnp.int32),
            jax.ShapeDtypeStruct((na, 1), jnp.float32),
        ],
    )(a8, xT8, u8, uincl, uexcl)
    return enc, adu


def _sc_select_gather(n, na, kk, hp, nf_table, enc):
    nw = _SC_CORES * _SC_SUBCORES
    apw = na // nw
    nv16 = n // 16

    mesh = plsc.VectorSubcoreMesh(core_axis_name="c", subcore_axis_name="s",
                                  num_cores=_SC_CORES,
                                  num_subcores=_SC_SUBCORES)

    @functools.partial(
        pl.kernel, mesh=mesh,
        compiler_params=pltpu.CompilerParams(needs_layout_passes=False),
        out_type=[
            jax.ShapeDtypeStruct((na * kk, hp), jnp.float32),
        ],
        scratch_types=[
            pltpu.VMEM((n,), jnp.int32),
            pltpu.VMEM((2, kk), jnp.int32),
            pltpu.VMEM((2, kk, hp), jnp.float32),
            pltpu.SemaphoreType.DMA,
            pltpu.SemaphoreType.DMA,
            pltpu.SemaphoreType.DMA,
            pltpu.SemaphoreType.DMA,
        ],
    )
    def sc_kernel(nf_hbm, enc_hbm, gfeat_out, encrow, idxa2, rows2,
                  gsa, gsb, wsa, wsb):
        wid = lax.axis_index("s") * _SC_CORES + lax.axis_index("c")
        base = wid * apw
        iota = lax.iota(jnp.int32, 16)
        gsem = [gsa, gsb]
        wsem = [wsa, wsb]

        def scan(la, s):
            pltpu.sync_copy(enc_hbm.at[la], encrow)
            idref = idxa2.at[s]

            def step(i, carry):
                for u in range(8):
                    off = i * 128 + u * 16
                    e = encrow[pl.ds(off, 16)]
                    m = e >= 0
                    plsc.store_scatter(idref, [e], iota + off, mask=m)
                return carry

            lax.fori_loop(0, n // 128, step, 0)

        def gather(s):
            pltpu.async_copy(nf_hbm.at[idxa2.at[s]], rows2.at[s], gsem[s])

        def write(la, s):
            pltpu.async_copy(rows2.at[s], gfeat_out.at[pl.ds(la * kk, kk)],
                             wsem[s])

        def drain_g(s):
            pltpu.make_async_copy(gfeat_out.at[pl.ds(0, kk)], rows2.at[s],
                                  gsem[s]).wait()

        def drain_w(s):
            pltpu.make_async_copy(rows2.at[s], gfeat_out.at[pl.ds(0, kk)],
                                  wsem[s]).wait()

        scan(base, 0)
        gather(0)
        scan(base + 1, 1)
        gather(1)
        drain_g(0)
        write(base, 0)

        def pair(j, carry):
            la = base + 2 * j
            scan(la, 0)
            drain_w(0)
            gather(0)
            drain_g(1)
            write(la - 1, 1)
            scan(la + 1, 1)
            drain_w(1)
            gather(1)
            drain_g(0)
            write(la, 0)
            return carry

        lax.fori_loop(1, apw // 2, pair, 0)
        drain_g(1)
        write(base + apw - 1, 1)
        drain_w(0)
        drain_w(1)

    return sc_kernel(nf_table, enc)[0]


def _edge_kernel(ablk, kk, hid, gfeat, aduc, a8b,
                 ew1a, w256, vA, vX, eb1, ew2, eb2, ew3, eb3,
                 gw1, gb1, gw2, gb2,
                 psum, inv_n,
                 glw1, glb1, glw2, glb2, glw3, glb3,
                 ow1a, ow1b, ob1, ow2, ob2, ow3, ob3,
                 out):
    gmean = jnp.sum(psum[...], axis=0, keepdims=True) * inv_n[0, 0]
    g = _gelu(_dot(gmean, glw1[...]) + glb1[...])
    g = _gelu(_dot(g, glw2[...]) + glb2[...])
    glob = _dot(g, glw3[...]) + glb3[...]

    gfb = gfeat[...]
    nduc = gfb[:, hid:hid + 1]
    ab = a8b[...]
    an2 = jnp.sum(ab * ab, axis=1, keepdims=True)
    d2cols = []
    for a in range(ablk):
        xs = gfb[a * kk:(a + 1) * kk]
        arow = ab[a:a + 1]
        ax = (xs[:, hid + 2:hid + 3] * arow[:, 0:1]
              + xs[:, hid + 3:hid + 4] * arow[:, 1:2]
              + xs[:, hid + 4:hid + 5] * arow[:, 2:3])
        d2cols.append(an2[a:a + 1] + xs[:, hid + 1:hid + 2] - 2.0 * ax)
    d2v = jnp.concatenate(d2cols, axis=0)
    x1 = (_dot(gfb[:, :hid], ew1a[...]) + eb1[...]
          + d2v * w256[...] + aduc[...] * vA[...] + nduc * vX[...])
    h = _gelu(x1)
    h = _gelu(_dot(h, ew2[...]) + eb2[...])
    eh = _dot(h, ew3[...]) + eb3[...]

    gt = _dot(_gelu(_dot(eh, gw1[...]) + gb1[...]), gw2[...]) + gb2[...]
    logits = gt - d2v

    aggs = []
    for a in range(ablk):
        lg = logits[a * kk:(a + 1) * kk]
        m = jnp.max(lg)
        w = jnp.exp(lg - m)
        w = w / jnp.sum(w)
        aggs.append(jnp.sum(w * eh[a * kk:(a + 1) * kk], axis=0, keepdims=True))
    agg = jnp.concatenate(aggs, axis=0)

    o = _gelu(_dot(agg, ow1a[...]) + _dot(glob, ow1b[...]) + ob1[...])
    o = _gelu(_dot(o, ow2[...]) + ob2[...])
    out[...] = _dot(o, ow3[...]) + ob3[...]


def kernel(input_coords, input_x, anchor_coords, flow_dir, params):
    x = input_coords[0]
    fin = input_x[0]
    a = anchor_coords[0]
    u = flow_dir[0]
    u = u / (jnp.linalg.norm(u) + 1e-8)

    n, cin = fin.shape
    na = a.shape[0]
    kk = min(128, n)

    a8 = jnp.zeros((na, 8), jnp.float32).at[:, :3].set(a)
    xT8 = jnp.zeros((8, n), jnp.float32).at[:3, :].set(x.T)
    u8 = jnp.zeros((8, 1), jnp.float32).at[:3, 0].set(u)

    x8 = jnp.zeros((n, 8), jnp.float32).at[:, :3].set(x)

    nblk = min(1024, n)
    nf_table, psum = _node_mlp(fin, x8, u8, params["node"], nblk)
    hid = nf_table.shape[1] - 128

    ablk2 = min(64, na)
    enc, adu = _distances(a8, xT8, u8, ablk2, kk)

    gfeat = _sc_select_gather(n, na, kk, hid + 128, nf_table, enc)

    aduc = jnp.broadcast_to(adu, (na, kk)).reshape(-1, 1)

    ablk3 = min(16, na)
    out = _edge_stage_fixed(gfeat, aduc, a8, psum, params, na, kk,
                            ablk3, n, hid)
    return out[None]


def _edge_stage_fixed(gfeat, aduc, a8, psum, params, na, kk, ablk, n, hid):
    hp = gfeat.shape[1]
    rows = ablk * kk
    grid = na // ablk

    (ew1, eb1), (ew2, eb2), (ew3, eb3) = params["edge"]
    (gw1, gb1), (gw2, gb2) = params["gate"]
    (glw1, glb1), (glw2, glb2), (glw3, glb3) = params["glob"]
    (ow1, ob1), (ow2, ob2), (ow3, ob3) = params["out"]

    ew1a = ew1[:hid]
    w256 = ew1[hid:hid + 1]
    vA = ew1[hid + 1:hid + 2] + ew1[hid + 3:hid + 4]
    vX = ew1[hid + 2:hid + 3] - ew1[hid + 3:hid + 4]
    ow1a, ow1b = ow1[:hid], ow1[hid:]
    inv_n = jnp.full((1, 1), 1.0 / n, jnp.float32)

    full = lambda arr: pl.BlockSpec(arr.shape, lambda i: tuple(0 for _ in arr.shape))

    def rowspec(w):
        return pl.BlockSpec((rows, w), lambda i: (i, 0))

    args = [gfeat, aduc, a8,
            ew1a, w256, vA, vX, eb1.reshape(1, -1), ew2, eb2.reshape(1, -1),
            ew3, eb3.reshape(1, -1),
            gw1, gb1.reshape(1, -1), gw2, gb2.reshape(1, -1),
            psum, inv_n,
            glw1, glb1.reshape(1, -1), glw2, glb2.reshape(1, -1), glw3,
            glb3.reshape(1, -1),
            ow1a, ow1b, ob1.reshape(1, -1), ow2, ob2.reshape(1, -1), ow3,
            ob3.reshape(1, -1)]
    in_specs = [rowspec(hp), rowspec(1),
                pl.BlockSpec((ablk, 8), lambda i: (i, 0))] + \
               [full(arr) for arr in args[3:]]

    out = pl.pallas_call(
        functools.partial(_edge_kernel, ablk, kk, hid),
        grid=(grid,),
        in_specs=in_specs,
        out_specs=pl.BlockSpec((ablk, hid), lambda i: (i, 0)),
        out_shape=jax.ShapeDtypeStruct((na, hid), jnp.float32),
    )(*args)
    return out


# speedup vs baseline: 1.0309x; 1.0309x over previous
import functools

import jax
import jax.numpy as jnp
from jax import lax
from jax.experimental import pallas as pl
from jax.experimental.pallas import tpu as pltpu
from jax.experimental.pallas import tpu_sc as plsc

_SC_CORES = 2
_SC_SUBCORES = 16


def _gelu(x):
    return 0.5 * x * (1.0 + jax.lax.erf(x * (2.0 ** -0.5)))


def _dot(a, b):
    return jax.lax.dot_general(a, b, (((1,), (0,)), ((), ())),
                               preferred_element_type=jnp.float32)


def _node_mlp_kernel(hid, fin, x8, u8, w1, b1, w2, b2, w3, b3, out, psum):
    h = _gelu(_dot(fin[...], w1[...]) + b1[...])
    h = _gelu(_dot(h, w2[...]) + b2[...])
    nf = _dot(h, w3[...]) + b3[...]
    nblk = nf.shape[0]
    x8b = x8[...]
    ndu = _dot(x8b, u8[...])
    xn2 = jnp.sum(x8b * x8b, axis=1, keepdims=True)
    pad = jnp.zeros((nblk, 123), jnp.float32)
    out[...] = jnp.concatenate([nf, ndu, xn2, x8b[:, :3], pad], axis=1)
    psum[...] = jnp.sum(nf, axis=0, keepdims=True).reshape(1, 1, hid)


def _node_mlp(fin, x8, u8, node_params, nblk):
    n, cin = fin.shape
    (w1, b1), (w2, b2), (w3, b3) = node_params
    hid = w3.shape[1]
    hp = hid + 128
    grid = n // nblk
    full = lambda *s: pl.BlockSpec(s, lambda i: tuple(0 for _ in s))
    out, psum = pl.pallas_call(
        functools.partial(_node_mlp_kernel, hid),
        grid=(grid,),
        in_specs=[
            pl.BlockSpec((nblk, cin), lambda i: (i, 0)),
            pl.BlockSpec((nblk, 8), lambda i: (i, 0)),
            full(8, 1),
            full(cin, hid), full(1, hid),
            full(hid, hid), full(1, hid),
            full(hid, hid), full(1, hid),
        ],
        out_specs=[
            pl.BlockSpec((nblk, hp), lambda i: (i, 0)),
            pl.BlockSpec((1, 1, hid), lambda i: (i, 0, 0)),
        ],
        out_shape=[
            jax.ShapeDtypeStruct((n, hp), jnp.float32),
            jax.ShapeDtypeStruct((grid, 1, hid), jnp.float32),
        ],
    )(fin, x8, u8, w1, b1.reshape(1, -1), w2, b2.reshape(1, -1), w3,
      b3.reshape(1, -1))
    return out, psum.reshape(grid, hid)


def _dist_kernel(kk, a8, xT8, u8, uincl, uexcl, enc, adu):
    ab = a8[...]
    xt = xT8[...]
    an2 = jnp.sum(ab * ab, axis=1, keepdims=True)
    xn2 = jnp.sum(xt * xt, axis=0, keepdims=True)
    prod = _dot(ab, xt)
    d2b = an2 + xn2 - 2.0 * prod
    adu[...] = _dot(ab, u8[...])

    bits = lax.bitcast_convert_type(d2b, jnp.uint32)
    sign = bits >> jnp.uint32(31)
    key = bits ^ (sign * jnp.uint32(0xFFFFFFFF) | jnp.uint32(0x80000000))
    p = jnp.zeros(d2b.shape[:1] + (1,), jnp.uint32)
    for b in range(31, -1, -1):
        cand = p | jnp.uint32((1 << b) - 1)
        cnt = jnp.sum((key <= cand).astype(jnp.int32), axis=1, keepdims=True)
        p = jnp.where(cnt >= kk, p, p | jnp.uint32(1 << b))
    vbits = jnp.where(p >= jnp.uint32(0x80000000),
                      p ^ jnp.uint32(0x80000000), ~p)
    thr_f = lax.bitcast_convert_type(vbits, jnp.float32)
    lt = d2b < thr_f
    eq = d2b == thr_f
    cnt_lt = jnp.sum(jnp.where(lt, 1.0, 0.0), axis=1, keepdims=True)
    budf = kk - cnt_lt

    n = d2b.shape[1]
    nch = n // 128
    ui = uincl[...]
    ue = uexcl[...]
    one = jnp.float32(1.0)
    zero = jnp.float32(0.0)
    peq = []
    for c in range(nch):
        sl = slice(c * 128, (c + 1) * 128)
        peq.append(_dot(jnp.where(eq[:, sl], one, zero), ui))
    tot_eq = jnp.concatenate([q[:, 127:128] for q in peq], axis=1)
    off_eq = _dot(tot_eq, ue)
    takes = []
    ptk = []
    for c in range(nch):
        sl = slice(c * 128, (c + 1) * 128)
        tie_rank = peq[c] + off_eq[:, c:c + 1]
        take_c = lt[:, sl] | (eq[:, sl] & (tie_rank <= budf))
        takes.append(take_c)
        ptk.append(_dot(jnp.where(take_c, one, zero), ui))
    tot_tk = jnp.concatenate([q[:, 127:128] for q in ptk], axis=1)
    off_tk = _dot(tot_tk, ue)
    for c in range(nch):
        pos = ptk[c] - 1.0 + off_tk[:, c:c + 1]
        enc[:, c * 128:(c + 1) * 128] = jnp.where(
            takes[c], pos.astype(jnp.int32), jnp.int32(-1))


def _distances(a8, xT8, u8, ablk, kk):
    na = a8.shape[0]
    n = xT8.shape[1]
    grid = na // ablk
    nch = n // 128
    ii = lax.broadcasted_iota(jnp.int32, (128, 128), 0)
    jj = lax.broadcasted_iota(jnp.int32, (128, 128), 1)
    uincl = jnp.where(ii <= jj, 1.0, 0.0).astype(jnp.float32)
    i2 = lax.broadcasted_iota(jnp.int32, (nch, nch), 0)
    j2 = lax.broadcasted_iota(jnp.int32, (nch, nch), 1)
    uexcl = jnp.where(i2 < j2, 1.0, 0.0).astype(jnp.float32)
    enc, adu = pl.pallas_call(
        functools.partial(_dist_kernel, kk),
        grid=(grid,),
        in_specs=[
            pl.BlockSpec((ablk, 8), lambda i: (i, 0)),
            pl.BlockSpec((8, n), lambda i: (0, 0)),
            pl.BlockSpec((8, 1), lambda i: (0, 0)),
            pl.BlockSpec((128, 128), lambda i: (0, 0)),
            pl.BlockSpec((nch, nch), lambda i: (0, 0)),
        ],
        out_specs=[
            pl.BlockSpec((ablk, n), lambda i: (i, 0)),
            pl.BlockSpec((ablk, 1), lambda i: (i, 0)),
        ],
        out_shape=[
            jax.ShapeDtypeStruct((na, n), jnp.int32),
            jax.ShapeDtypeStruct((na, 1), jnp.float32),
        ],
    )(a8, xT8, u8, uincl, uexcl)
    return enc, adu


def _sc_select_gather(n, na, kk, hp, nf_table, enc):
    nw = _SC_CORES * _SC_SUBCORES
    apw = na // nw
    nv16 = n // 16

    mesh = plsc.VectorSubcoreMesh(core_axis_name="c", subcore_axis_name="s",
                                  num_cores=_SC_CORES,
                                  num_subcores=_SC_SUBCORES)

    @functools.partial(
        pl.kernel, mesh=mesh,
        compiler_params=pltpu.CompilerParams(needs_layout_passes=False),
        out_type=[
            jax.ShapeDtypeStruct((na * kk, hp), jnp.float32),
        ],
        scratch_types=[
            pltpu.VMEM((n,), jnp.int32),
            pltpu.VMEM((2, kk), jnp.int32),
            pltpu.VMEM((2, kk, hp), jnp.float32),
            pltpu.SemaphoreType.DMA,
            pltpu.SemaphoreType.DMA,
            pltpu.SemaphoreType.DMA,
            pltpu.SemaphoreType.DMA,
        ],
    )
    def sc_kernel(nf_hbm, enc_hbm, gfeat_out, encrow, idxa2, rows2,
                  gsa, gsb, wsa, wsb):
        wid = lax.axis_index("s") * _SC_CORES + lax.axis_index("c")
        base = wid * apw
        iota = lax.iota(jnp.int32, 16)
        gsem = [gsa, gsb]
        wsem = [wsa, wsb]

        def scan(la, s):
            pltpu.sync_copy(enc_hbm.at[la], encrow)
            idref = idxa2.at[s]

            def step(i, carry):
                for u in range(8):
                    off = i * 128 + u * 16
                    e = encrow[pl.ds(off, 16)]
                    m = e >= 0
                    plsc.store_scatter(idref, [e], iota + off, mask=m)
                return carry

            lax.fori_loop(0, n // 128, step, 0)

        def gather(s):
            pltpu.async_copy(nf_hbm.at[idxa2.at[s]], rows2.at[s], gsem[s])

        def write(la, s):
            pltpu.async_copy(rows2.at[s], gfeat_out.at[pl.ds(la * kk, kk)],
                             wsem[s])

        def drain_g(s):
            pltpu.make_async_copy(gfeat_out.at[pl.ds(0, kk)], rows2.at[s],
                                  gsem[s]).wait()

        def drain_w(s):
            pltpu.make_async_copy(rows2.at[s], gfeat_out.at[pl.ds(0, kk)],
                                  wsem[s]).wait()

        scan(base, 0)
        gather(0)
        scan(base + 1, 1)
        gather(1)
        drain_g(0)
        write(base, 0)

        def pair(j, carry):
            la = base + 2 * j
            scan(la, 0)
            drain_w(0)
            gather(0)
            drain_g(1)
            write(la - 1, 1)
            scan(la + 1, 1)
            drain_w(1)
            gather(1)
            drain_g(0)
            write(la, 0)
            return carry

        lax.fori_loop(1, apw // 2, pair, 0)
        drain_g(1)
        write(base + apw - 1, 1)
        drain_w(0)
        drain_w(1)

    return sc_kernel(nf_table, enc)[0]


def _edge_kernel(ablk, kk, hid, gfeat, aduc, a8b,
                 ew1a, w256, vA, vX, eb1, ew2, eb2, ew3, eb3,
                 gw1, gb1, gw2, gb2,
                 psum, inv_n,
                 glw1, glb1, glw2, glb2, glw3, glb3,
                 ow1a, ow1b, ob1, ow2, ob2, ow3, ob3,
                 out):
    gmean = jnp.sum(psum[...], axis=0, keepdims=True) * inv_n[0, 0]
    g = _gelu(_dot(gmean, glw1[...]) + glb1[...])
    g = _gelu(_dot(g, glw2[...]) + glb2[...])
    glob = _dot(g, glw3[...]) + glb3[...]

    bf16 = jnp.bfloat16
    gfb = gfeat[...]
    nduc = gfb[:, hid:hid + 1]
    ab = a8b[...]
    an2 = jnp.sum(ab * ab, axis=1, keepdims=True)
    d2cols = []
    for a in range(ablk):
        xs = gfb[a * kk:(a + 1) * kk]
        arow = ab[a:a + 1]
        ax = (xs[:, hid + 2:hid + 3] * arow[:, 0:1]
              + xs[:, hid + 3:hid + 4] * arow[:, 1:2]
              + xs[:, hid + 4:hid + 5] * arow[:, 2:3])
        d2cols.append(an2[a:a + 1] + xs[:, hid + 1:hid + 2] - 2.0 * ax)
    d2v = jnp.concatenate(d2cols, axis=0)
    x1 = (_dot(gfb[:, :hid].astype(bf16), ew1a[...]) + eb1[...]
          + d2v * w256[...] + aduc[...] * vA[...] + nduc * vX[...])
    h = _gelu(x1)
    h = _gelu(_dot(h.astype(bf16), ew2[...]) + eb2[...])
    eh = _dot(h.astype(bf16), ew3[...]) + eb3[...]

    gt = _dot(_gelu(_dot(eh.astype(bf16), gw1[...]) + gb1[...]),
              gw2[...]) + gb2[...]
    logits = gt - d2v

    aggs = []
    for a in range(ablk):
        lg = logits[a * kk:(a + 1) * kk]
        m = jnp.max(lg)
        w = jnp.exp(lg - m)
        w = w / jnp.sum(w)
        aggs.append(jnp.sum(w * eh[a * kk:(a + 1) * kk], axis=0, keepdims=True))
    agg = jnp.concatenate(aggs, axis=0)

    o = _gelu(_dot(agg, ow1a[...]) + _dot(glob, ow1b[...]) + ob1[...])
    o = _gelu(_dot(o, ow2[...]) + ob2[...])
    out[...] = _dot(o, ow3[...]) + ob3[...]


def kernel(input_coords, input_x, anchor_coords, flow_dir, params):
    x = input_coords[0]
    fin = input_x[0]
    a = anchor_coords[0]
    u = flow_dir[0]
    u = u / (jnp.linalg.norm(u) + 1e-8)

    n, cin = fin.shape
    na = a.shape[0]
    kk = min(128, n)

    a8 = jnp.zeros((na, 8), jnp.float32).at[:, :3].set(a)
    xT8 = jnp.zeros((8, n), jnp.float32).at[:3, :].set(x.T)
    u8 = jnp.zeros((8, 1), jnp.float32).at[:3, 0].set(u)

    x8 = jnp.zeros((n, 8), jnp.float32).at[:, :3].set(x)

    nblk = min(1024, n)
    nf_table, psum = _node_mlp(fin, x8, u8, params["node"], nblk)
    hid = nf_table.shape[1] - 128

    ablk2 = min(64, na)
    enc, adu = _distances(a8, xT8, u8, ablk2, kk)

    gfeat = _sc_select_gather(n, na, kk, hid + 128, nf_table, enc)

    aduc = jnp.broadcast_to(adu, (na, kk)).reshape(-1, 1)

    ablk3 = min(16, na)
    out = _edge_stage_fixed(gfeat, aduc, a8, psum, params, na, kk,
                            ablk3, n, hid)
    return out[None]


def _edge_stage_fixed(gfeat, aduc, a8, psum, params, na, kk, ablk, n, hid):
    hp = gfeat.shape[1]
    rows = ablk * kk
    grid = na // ablk

    (ew1, eb1), (ew2, eb2), (ew3, eb3) = params["edge"]
    (gw1, gb1), (gw2, gb2) = params["gate"]
    (glw1, glb1), (glw2, glb2), (glw3, glb3) = params["glob"]
    (ow1, ob1), (ow2, ob2), (ow3, ob3) = params["out"]

    ew1a = ew1[:hid].astype(jnp.bfloat16)
    w256 = ew1[hid:hid + 1]
    vA = ew1[hid + 1:hid + 2] + ew1[hid + 3:hid + 4]
    vX = ew1[hid + 2:hid + 3] - ew1[hid + 3:hid + 4]
    ew2 = ew2.astype(jnp.bfloat16)
    ew3 = ew3.astype(jnp.bfloat16)
    gw1 = gw1.astype(jnp.bfloat16)
    ow1a, ow1b = ow1[:hid], ow1[hid:]
    inv_n = jnp.full((1, 1), 1.0 / n, jnp.float32)

    full = lambda arr: pl.BlockSpec(arr.shape, lambda i: tuple(0 for _ in arr.shape))

    def rowspec(w):
        return pl.BlockSpec((rows, w), lambda i: (i, 0))

    args = [gfeat, aduc, a8,
            ew1a, w256, vA, vX, eb1.reshape(1, -1), ew2, eb2.reshape(1, -1),
            ew3, eb3.reshape(1, -1),
            gw1, gb1.reshape(1, -1), gw2, gb2.reshape(1, -1),
            psum, inv_n,
            glw1, glb1.reshape(1, -1), glw2, glb2.reshape(1, -1), glw3,
            glb3.reshape(1, -1),
            ow1a, ow1b, ob1.reshape(1, -1), ow2, ob2.reshape(1, -1), ow3,
            ob3.reshape(1, -1)]
    in_specs = [rowspec(hp), rowspec(1),
                pl.BlockSpec((ablk, 8), lambda i: (i, 0))] + \
               [full(arr) for arr in args[3:]]

    out = pl.pallas_call(
        functools.partial(_edge_kernel, ablk, kk, hid),
        grid=(grid,),
        in_specs=in_specs,
        out_specs=pl.BlockSpec((ablk, hid), lambda i: (i, 0)),
        out_shape=jax.ShapeDtypeStruct((na, hid), jnp.float32),
    )(*args)
    return out
